# baseline (speedup 1.0000x reference)
import functools

import jax
import jax.numpy as jnp
from jax import lax
from jax.experimental import pallas as pl
from jax.experimental.pallas import tpu as pltpu
from jax.experimental.pallas import tpu_sc as plsc

_HIDDEN = 256
_LATENT = 32
_K = 512
_RES_N = 2


_T = 16
_ROWS_BLK = 512


def _exact_tree_sum(sq):
    out = None
    for h in (0, 1):
        half = sq[:, 128 * h:128 * h + 128]
        s = half[:, 0:8]
        for j in range(1, 16):
            s = s + half[:, 8 * j:8 * j + 8]
        v = s[:, 0:4] + s[:, 4:8]
        v = v[:, 0:2] + v[:, 2:4]
        v = v[:, 0:1] + v[:, 1:2]
        out = v if out is None else out + v
    return out


def _vq_argmin_body(zf_ref, cbt_ref, cb_ref, idx_ref):
    zf = zf_ref[...]
    cbt = cbt_ref[...]
    cb = cb_ref[...]
    r = zf.shape[0]

    s = jnp.dot(zf, cbt, preferred_element_type=jnp.float32,
                precision=lax.Precision.HIGHEST)
    cbn = jnp.sum(cbt * cbt, axis=0, keepdims=True)
    u = cbn - 2.0 * s
    k_iota = lax.broadcasted_iota(jnp.int32, u.shape, 1)

    cands, dexact = [], []
    for _t in range(_T):
        m = jnp.min(u, axis=1, keepdims=True)
        km = jnp.min(jnp.where(u == m, k_iota, jnp.int32(_K)),
                     axis=1, keepdims=True)
        u = jnp.where(k_iota == km, jnp.float32(jnp.inf), u)
        onehot = (k_iota == km).astype(jnp.float32)
        row = jnp.dot(onehot, cb, preferred_element_type=jnp.float32,
                      precision=lax.Precision.HIGHEST)
        diff = zf - row
        dexact.append(_exact_tree_sum(diff * diff))
        cands.append(km)
    dmat = jnp.concatenate(dexact, axis=1)
    cmat = jnp.concatenate(cands, axis=1)

    md = jnp.min(dmat, axis=1, keepdims=True)
    idx_ref[...] = jnp.min(jnp.where(dmat == md, cmat, jnp.int32(_K)), axis=1)


def _vq_argmin(zf, cb, cbt):
    n, c = zf.shape
    return pl.pallas_call(
        _vq_argmin_body,
        grid=(n // _ROWS_BLK,),
        in_specs=[
            pl.BlockSpec((_ROWS_BLK, c), lambda i: (i, 0)),
            pl.BlockSpec((c, _K), lambda i: (0, 0)),
            pl.BlockSpec((_K, c), lambda i: (0, 0)),
        ],
        out_specs=pl.BlockSpec((_ROWS_BLK,), lambda i: (i,)),
        out_shape=jax.ShapeDtypeStruct((n,), jnp.int32),
    )(zf, cbt, cb)


_NC = 2
_NS = 16
_NW = _NC * _NS


def _make_sc_gather(b, hw):
    w_per_b = _NW // b
    cols = hw // w_per_b
    mesh = plsc.VectorSubcoreMesh(core_axis_name="c", subcore_axis_name="s")

    @functools.partial(
        pl.kernel, mesh=mesh,
        out_type=jax.ShapeDtypeStruct((b, hw, _HIDDEN), jnp.float32),
        scratch_types=[
            pltpu.VMEM((cols,), jnp.int32),
            pltpu.VMEM((cols, _HIDDEN), jnp.float32),
            pltpu.SemaphoreType.DMA,
        ],
    )
    def gather(table_hbm, idx_hbm, out_hbm, idx_v, rows_v, sem):
        wid = lax.axis_index("s") * _NC + lax.axis_index("c")
        bi = wid // w_per_b
        col = (wid % w_per_b) * cols
        pltpu.sync_copy(idx_hbm.at[bi, pl.ds(col, cols)], idx_v)
        pltpu.async_copy(table_hbm.at[idx_v], rows_v, sem).wait()
        pltpu.sync_copy(rows_v, out_hbm.at[bi, pl.ds(col, cols)])

    return gather


def _conv(x, w, b, stride, pad):
    y = lax.conv_general_dilated(x, w, (stride, stride), [(pad, pad), (pad, pad)],
                                 dimension_numbers=('NCHW', 'OIHW', 'NCHW'))
    return y + b[None, :, None, None]


def _deconv(x, w, b, stride, pad):
    wt = jnp.transpose(jnp.flip(w, axis=(2, 3)), (1, 0, 2, 3))
    pp = w.shape[2] - 1 - pad
    y = lax.conv_general_dilated(x, wt, (1, 1), [(pp, pp), (pp, pp)],
                                 lhs_dilation=(stride, stride),
                                 dimension_numbers=('NCHW', 'OIHW', 'NCHW'))
    return y + b[None, :, None, None]


def _res(h, w1, b1, w2, b2):
    r = jax.nn.relu(h)
    r = _conv(r, w1, b1, 1, 1)
    r = jax.nn.relu(r)
    r = _conv(r, w2, b2, 1, 0)
    return h + r


def _enc(x, p):
    h = jax.nn.relu(_conv(x, p['enc_w0'], p['enc_b0'], 2, 1))
    h = jax.nn.relu(_conv(h, p['enc_w1'], p['enc_b1'], 2, 1))
    h = _conv(h, p['enc_w2'], p['enc_b2'], 1, 1)
    for i in range(_RES_N):
        h = _res(h, p['enc_r%d_w1' % i], p['enc_r%d_b1' % i],
                 p['enc_r%d_w2' % i], p['enc_r%d_b2' % i])
    return h


def _dec(zq, p):
    h = _conv(zq, p['dec_w0'], p['dec_b0'], 1, 1)
    for i in range(_RES_N):
        h = _res(h, p['dec_r%d_w1' % i], p['dec_r%d_b1' % i],
                 p['dec_r%d_w2' % i], p['dec_r%d_b2' % i])
    h = jax.nn.relu(_deconv(h, p['dec_tw0'], p['dec_tb0'], 2, 1))
    h = _deconv(h, p['dec_tw1'], p['dec_tb1'], 2, 1)
    return h


def kernel(x, params):
    p = params
    z = _enc(x, p)
    B, C, H, W = z.shape
    n = B * H * W
    zf = jnp.transpose(z.reshape(B, C, H * W), (0, 2, 1)).reshape(n, C)
    cb = p['code_books']
    idx = _vq_argmin(lax.stop_gradient(zf), cb, cb.T).reshape(B, H * W)
    zq_flat = _make_sc_gather(B, H * W)(cb, idx)
    zq = jnp.transpose(zq_flat.reshape(B, _LATENT, _LATENT, C), (0, 3, 1, 2))
    z_st = zq + lax.stop_gradient(z - zq)
    x_pred = _dec(zq, p)
    return (x_pred, z_st, zq)


# speedup vs baseline: 1.5052x; 1.5052x over previous
import functools

import jax
import jax.numpy as jnp
from jax import lax
from jax.experimental import pallas as pl
from jax.experimental.pallas import tpu as pltpu
from jax.experimental.pallas import tpu_sc as plsc

_HIDDEN = 256
_LATENT = 32
_K = 512
_RES_N = 2


_T = 8
_ROWS_BLK = 512


def _exact_tree_sum(sq):
    out = None
    for h in (0, 1):
        half = sq[:, 128 * h:128 * h + 128]
        s = half[:, 0:8]
        for j in range(1, 16):
            s = s + half[:, 8 * j:8 * j + 8]
        v = s[:, 0:4] + s[:, 4:8]
        v = v[:, 0:2] + v[:, 2:4]
        v = v[:, 0:1] + v[:, 1:2]
        out = v if out is None else out + v
    return out


def _vq_argmin_body(zf_ref, cbt_ref, cb_ref, idx_ref):
    zf = zf_ref[...]
    cbt = cbt_ref[...]
    cb = cb_ref[...]
    r = zf.shape[0]

    s = jnp.dot(zf, cbt, preferred_element_type=jnp.float32,
                precision=lax.Precision.HIGHEST)
    cbn = jnp.sum(cbt * cbt, axis=0, keepdims=True)
    u = cbn - 2.0 * s
    k_iota = lax.broadcasted_iota(jnp.int32, u.shape, 1)

    cands, dexact = [], []
    for _t in range(_T):
        m = jnp.min(u, axis=1, keepdims=True)
        km = jnp.min(jnp.where(u == m, k_iota, jnp.int32(_K)),
                     axis=1, keepdims=True)
        u = jnp.where(k_iota == km, jnp.float32(jnp.inf), u)
        onehot = (k_iota == km).astype(jnp.float32)
        row = jnp.dot(onehot, cb, preferred_element_type=jnp.float32,
                      precision=lax.Precision.HIGHEST)
        diff = zf - row
        dexact.append(_exact_tree_sum(diff * diff))
        cands.append(km)
    dmat = jnp.concatenate(dexact, axis=1)
    cmat = jnp.concatenate(cands, axis=1)

    md = jnp.min(dmat, axis=1, keepdims=True)
    idx_ref[...] = jnp.min(jnp.where(dmat == md, cmat, jnp.int32(_K)), axis=1)


def _vq_argmin(zf, cb, cbt):
    n, c = zf.shape
    return pl.pallas_call(
        _vq_argmin_body,
        grid=(n // _ROWS_BLK,),
        in_specs=[
            pl.BlockSpec((_ROWS_BLK, c), lambda i: (i, 0)),
            pl.BlockSpec((c, _K), lambda i: (0, 0)),
            pl.BlockSpec((_K, c), lambda i: (0, 0)),
        ],
        out_specs=pl.BlockSpec((_ROWS_BLK,), lambda i: (i,)),
        out_shape=jax.ShapeDtypeStruct((n,), jnp.int32),
    )(zf, cbt, cb)


_NC = 2
_NS = 16
_NW = _NC * _NS


def _make_sc_gather(b, hw):
    w_per_b = _NW // b
    cols = hw // w_per_b
    mesh = plsc.VectorSubcoreMesh(core_axis_name="c", subcore_axis_name="s")

    @functools.partial(
        pl.kernel, mesh=mesh,
        out_type=jax.ShapeDtypeStruct((b, hw, _HIDDEN), jnp.float32),
        scratch_types=[
            pltpu.VMEM((cols,), jnp.int32),
            pltpu.VMEM((cols, _HIDDEN), jnp.float32),
            pltpu.SemaphoreType.DMA,
        ],
    )
    def gather(table_hbm, idx_hbm, out_hbm, idx_v, rows_v, sem):
        wid = lax.axis_index("s") * _NC + lax.axis_index("c")
        bi = wid // w_per_b
        col = (wid % w_per_b) * cols
        pltpu.sync_copy(idx_hbm.at[bi, pl.ds(col, cols)], idx_v)
        pltpu.async_copy(table_hbm.at[idx_v], rows_v, sem).wait()
        pltpu.sync_copy(rows_v, out_hbm.at[bi, pl.ds(col, cols)])

    return gather


def _conv(x, w, b, stride, pad):
    y = lax.conv_general_dilated(x, w, (stride, stride), [(pad, pad), (pad, pad)],
                                 dimension_numbers=('NCHW', 'OIHW', 'NCHW'))
    return y + b[None, :, None, None]


def _deconv(x, w, b, stride, pad):
    wt = jnp.transpose(jnp.flip(w, axis=(2, 3)), (1, 0, 2, 3))
    pp = w.shape[2] - 1 - pad
    y = lax.conv_general_dilated(x, wt, (1, 1), [(pp, pp), (pp, pp)],
                                 lhs_dilation=(stride, stride),
                                 dimension_numbers=('NCHW', 'OIHW', 'NCHW'))
    return y + b[None, :, None, None]


def _res(h, w1, b1, w2, b2):
    r = jax.nn.relu(h)
    r = _conv(r, w1, b1, 1, 1)
    r = jax.nn.relu(r)
    r = _conv(r, w2, b2, 1, 0)
    return h + r


def _enc(x, p):
    h = jax.nn.relu(_conv(x, p['enc_w0'], p['enc_b0'], 2, 1))
    h = jax.nn.relu(_conv(h, p['enc_w1'], p['enc_b1'], 2, 1))
    h = _conv(h, p['enc_w2'], p['enc_b2'], 1, 1)
    for i in range(_RES_N):
        h = _res(h, p['enc_r%d_w1' % i], p['enc_r%d_b1' % i],
                 p['enc_r%d_w2' % i], p['enc_r%d_b2' % i])
    return h


def _dec(zq, p):
    h = _conv(zq, p['dec_w0'], p['dec_b0'], 1, 1)
    for i in range(_RES_N):
        h = _res(h, p['dec_r%d_w1' % i], p['dec_r%d_b1' % i],
                 p['dec_r%d_w2' % i], p['dec_r%d_b2' % i])
    h = jax.nn.relu(_deconv(h, p['dec_tw0'], p['dec_tb0'], 2, 1))
    h = _deconv(h, p['dec_tw1'], p['dec_tb1'], 2, 1)
    return h


def kernel(x, params):
    p = params
    z = _enc(x, p)
    B, C, H, W = z.shape
    n = B * H * W
    zf = jnp.transpose(z.reshape(B, C, H * W), (0, 2, 1)).reshape(n, C)
    cb = p['code_books']
    idx = _vq_argmin(lax.stop_gradient(zf), cb, cb.T).reshape(B, H * W)
    zq_flat = _make_sc_gather(B, H * W)(cb, idx)
    zq = jnp.transpose(zq_flat.reshape(B, _LATENT, _LATENT, C), (0, 3, 1, 2))
    z_st = zq + lax.stop_gradient(z - zq)
    x_pred = _dec(zq, p)
    return (x_pred, z_st, zq)
